# Initial kernel scaffold; baseline (speedup 1.0000x reference)
import jax
import jax.numpy as jnp
from jax.experimental import pallas as pl


def kernel(data, data1, data2, data3, output1, output2, output3, output):
    raise NotImplementedError("write your pallas kernel here")


# speedup vs baseline: 17.2449x; 17.2449x over previous
import jax
import jax.numpy as jnp
from jax.experimental import pallas as pl
from jax.experimental.pallas import tpu as pltpu

_STEP = 0.175
_NBINS = 128
_ROWS_PER_BLOCK = 8
_MIN_BLOCK = 512


def _min_kernel(d1_ref, d2_ref, d3_ref, o_ref):
    lane = jax.lax.broadcasted_iota(jnp.int32, (1, 128), 1)
    m1 = jnp.min(d1_ref[...])
    m2 = jnp.min(d2_ref[...])
    m3 = jnp.min(d3_ref[...])
    v = jnp.where(lane == 0, m1,
                  jnp.where(lane == 1, m2,
                            jnp.where(lane == 2, m3, jnp.float32(0))))
    o_ref[...] = v.reshape(1, 1, 128)


def _bin_codes(x, lower):
    idx = jnp.ceil((x - lower) / _STEP) - 1.0
    return jnp.clip(idx, 0.0, float(_NBINS - 1)).astype(jnp.int32)


def _xlogx(h):
    return h * jnp.log(jnp.maximum(h, 1.0))


def _main_kernel(d1_ref, d2_ref, d3_ref, o1_ref, o2_ref, o3_ref,
                 data_ref, outp_ref, lowers_ref, o_ref):
    D = d1_ref.shape[1]
    R = d1_ref.shape[0]
    f32 = jnp.float32

    mse_p = jnp.sum((data_ref[...] - outp_ref[...]) ** 2)

    i1 = _bin_codes(d1_ref[...], lowers_ref[0])
    i2 = _bin_codes(d2_ref[...], lowers_ref[1])
    i3 = _bin_codes(d3_ref[...], lowers_ref[2])
    t123 = (i1 * _NBINS + i2) * _NBINS + i3

    i2T = i2.T
    i3T = i3.T
    tT = t123.T

    row_iota = jax.lax.broadcasted_iota(jnp.int32, (_NBINS, D), 0)
    col_iota = jax.lax.broadcasted_iota(jnp.int32, (D, _NBINS), 1)

    acc1 = jnp.zeros((_NBINS, 1), f32)
    acc2 = jnp.zeros((1, _NBINS), f32)
    acc3 = jnp.zeros((1, _NBINS), f32)
    acc12 = jnp.zeros((1, _NBINS), f32)
    acc13 = jnp.zeros((1, _NBINS), f32)
    acc23 = jnp.zeros((1, _NBINS), f32)
    acc123 = jnp.zeros((1, 128), f32)

    for r in range(R):
        ohT1 = jnp.where(row_iota == i1[r:r + 1, :], 1.0, 0.0).astype(jnp.bfloat16)
        ohT2 = jnp.where(row_iota == i2[r:r + 1, :], 1.0, 0.0).astype(jnp.bfloat16)
        oh2 = jnp.where(col_iota == i2T[:, r:r + 1], 1.0, 0.0).astype(jnp.bfloat16)
        oh3 = jnp.where(col_iota == i3T[:, r:r + 1], 1.0, 0.0).astype(jnp.bfloat16)

        h12 = jnp.dot(ohT1, oh2, preferred_element_type=f32)
        h13 = jnp.dot(ohT1, oh3, preferred_element_type=f32)
        h23 = jnp.dot(ohT2, oh3, preferred_element_type=f32)

        acc12 = acc12 + jnp.sum(_xlogx(h12), axis=0, keepdims=True)
        acc13 = acc13 + jnp.sum(_xlogx(h13), axis=0, keepdims=True)
        acc23 = acc23 + jnp.sum(_xlogx(h23), axis=0, keepdims=True)
        acc1 = acc1 + _xlogx(jnp.sum(h12, axis=1, keepdims=True))
        acc2 = acc2 + _xlogx(jnp.sum(h12, axis=0, keepdims=True))
        acc3 = acc3 + _xlogx(jnp.sum(h13, axis=0, keepdims=True))

        tcol = tT[:, r:r + 1]
        for c in range(D // 128):
            trow_c = t123[r:r + 1, c * 128:(c + 1) * 128]
            eq = jnp.where(tcol == trow_c, 1.0, 0.0)
            cnt = jnp.sum(eq, axis=0, keepdims=True)
            acc123 = acc123 + jnp.log(cnt)

    s1 = jnp.sum(acc1)
    s2 = jnp.sum(acc2)
    s3 = jnp.sum(acc3)
    s12 = jnp.sum(acc12)
    s13 = jnp.sum(acc13)
    s23 = jnp.sum(acc23)
    s123 = jnp.sum(acc123)

    ds = (d1_ref[...], d2_ref[...], d3_ref[...])
    os_ = (o1_ref[...], o2_ref[...], o3_ref[...])
    m = [jnp.max(d, axis=1, keepdims=True) for d in ds]
    e = [jnp.exp(ds[a] - m[a]) for a in range(3)]
    s = [jnp.sum(e[a], axis=1, keepdims=True) for a in range(3)]
    U = [jnp.sum(e[a] * os_[a], axis=1, keepdims=True) for a in range(3)]
    V = [jnp.sum(e[a] * ds[a], axis=1, keepdims=True) for a in range(3)]
    mo = [jnp.max(o, axis=1, keepdims=True) for o in os_]
    so = [jnp.sum(jnp.exp(os_[a] - mo[a]), axis=1, keepdims=True) for a in range(3)]

    def _xc(C):
        mstar = m[C[0]]
        mostar = mo[C[0]]
        for a in C[1:]:
            mstar = jnp.maximum(mstar, m[a])
            mostar = jnp.maximum(mostar, mo[a])
        Z = sum(jnp.exp(m[a] - mstar) * s[a] for a in C)
        TO = sum(jnp.exp(m[a] - mstar) * U[a] for a in C) / Z
        TD = sum(jnp.exp(m[a] - mstar) * V[a] for a in C) / Z
        Zo = sum(jnp.exp(mo[a] - mostar) * so[a] for a in C)
        lse_d = mstar + jnp.log(Z)
        lse_o = mostar + jnp.log(Zo)
        P = TO - lse_o
        T = TD - lse_d
        nC = len(C) * D
        return jnp.sum(-P - (T - P) / nC)

    x1 = _xc((0,))
    x2 = _xc((1,))
    x3 = _xc((2,))
    x13 = _xc((0, 2))
    x23 = _xc((1, 2))
    x12 = _xc((0, 1))
    x123 = _xc((0, 1, 2))

    vals = (mse_p, s1, s2, s3, s12, s13, s23, s123,
            x1, x2, x3, x13, x23, x12, x123)
    lane = jax.lax.broadcasted_iota(jnp.int32, (1, 128), 1)
    out_v = jnp.zeros((1, 128), f32)
    for k, v in enumerate(vals):
        out_v = jnp.where(lane == k, v, out_v)
    o_ref[...] = out_v.reshape(1, 1, 128)


def kernel(data, data1, data2, data3, output1, output2, output3, output):
    B, D = data1.shape
    f32 = jnp.float32

    mb = min(_MIN_BLOCK, B)
    nb1 = B // mb
    mins = pl.pallas_call(
        _min_kernel,
        out_shape=jax.ShapeDtypeStruct((nb1, 1, 128), f32),
        grid=(nb1,),
        in_specs=[pl.BlockSpec((mb, D), lambda i: (i, 0))] * 3,
        out_specs=pl.BlockSpec((1, 1, 128), lambda i: (i, 0, 0)),
        compiler_params=pltpu.CompilerParams(
            dimension_semantics=("parallel",)),
        name="sae_loss_mins",
    )(data1, data2, data3)
    lowers = jnp.floor(jnp.min(mins[:, 0, :3], axis=0))

    R = _ROWS_PER_BLOCK
    nb2 = B // R
    part = pl.pallas_call(
        _main_kernel,
        out_shape=jax.ShapeDtypeStruct((nb2, 1, 128), f32),
        grid=(nb2,),
        in_specs=[
            pl.BlockSpec((R, D), lambda i: (i, 0)),
            pl.BlockSpec((R, D), lambda i: (i, 0)),
            pl.BlockSpec((R, D), lambda i: (i, 0)),
            pl.BlockSpec((R, D), lambda i: (i, 0)),
            pl.BlockSpec((R, D), lambda i: (i, 0)),
            pl.BlockSpec((R, D), lambda i: (i, 0)),
            pl.BlockSpec((R, 3 * D), lambda i: (i, 0)),
            pl.BlockSpec((R, 3 * D), lambda i: (i, 0)),
            pl.BlockSpec(memory_space=pltpu.SMEM),
        ],
        out_specs=pl.BlockSpec((1, 1, 128), lambda i: (i, 0, 0)),
        compiler_params=pltpu.CompilerParams(
            dimension_semantics=("parallel",),
            vmem_limit_bytes=56 * 1024 * 1024),
        name="sae_loss_main",
    )(data1, data2, data3, output1, output2, output3, data, output, lowers)

    sums = jnp.sum(part[:, 0, :], axis=0)
    (mse_s, s1, s2, s3, s12, s13, s23, s123,
     x1, x2, x3, x13, x23, x12, x123) = [sums[k] for k in range(15)]

    n = f32(D)
    logn = jnp.log(n)
    Bf = f32(B)

    mse = 0.5 * mse_s / (Bf * 3 * n)
    H_d1 = logn - s1 / (Bf * n)
    H_d2 = logn - s2 / (Bf * n)
    H_d3 = logn - s3 / (Bf * n)
    H_in13 = logn - s13 / (Bf * n)
    H_in23 = logn - s23 / (Bf * n)
    H_in12 = logn - s12 / (Bf * n)

    H_o1 = x1 / Bf
    H_o2 = x2 / Bf
    H_o3 = x3 / Bf
    H_o13 = x13 / Bf
    H_o23 = x23 / Bf
    H_o12 = x12 / Bf
    H_o123 = x123 / Bf

    H_1 = H_d1 - H_o1
    H_2 = H_d2 - H_o2
    H_3 = H_d3 - H_o3

    H_MI13 = (H_o1 + H_o3 - H_o13) - (H_d1 + H_d3 - H_in13)
    H_MI23 = (H_o2 + H_o3 - H_o23) - (H_d2 + H_d3 - H_in23)
    H_MI12 = (H_o1 + H_o2 - H_o12) - (H_d1 + H_d2 - H_in12)

    data_mu = (s3 + s123 - s13 - s23) / n
    label_cmi = H_o23 - H_o3 + H_o13 - H_o123
    CMI = label_cmi - data_mu

    return 0.9 * mse + 0.1 * (H_1 ** 2 + H_2 ** 2 + H_3 ** 2
                              + H_MI13 ** 2 + H_MI23 ** 2 + H_MI12 ** 2
                              + CMI ** 2)


# speedup vs baseline: 25.5155x; 1.4796x over previous
import jax
import jax.numpy as jnp
from jax.experimental import pallas as pl
from jax.experimental.pallas import tpu as pltpu

_STEP = 0.175
_NBINS = 128
_ROWS_PER_BLOCK = 8
_MIN_BLOCK = 512


def _min_kernel(d1_ref, d2_ref, d3_ref, o_ref):
    lane = jax.lax.broadcasted_iota(jnp.int32, (1, 128), 1)
    m1 = jnp.min(d1_ref[...])
    m2 = jnp.min(d2_ref[...])
    m3 = jnp.min(d3_ref[...])
    v = jnp.where(lane == 0, m1,
                  jnp.where(lane == 1, m2,
                            jnp.where(lane == 2, m3, jnp.float32(0))))
    o_ref[...] = v.reshape(1, 1, 128)


def _bin_codes(x, lower):
    idx = jnp.ceil((x - lower) / _STEP) - 1.0
    return jnp.clip(idx, 0.0, float(_NBINS - 1)).astype(jnp.int32)


def _xlogx(h):
    return h * jnp.log(jnp.maximum(h, 1.0))


def _main_kernel(d1_ref, d2_ref, d3_ref, o1_ref, o2_ref, o3_ref,
                 data_ref, outp_ref, lowers_ref, o_ref):
    D = d1_ref.shape[1]
    R = d1_ref.shape[0]
    f32 = jnp.float32

    mse_p = jnp.sum((data_ref[...] - outp_ref[...]) ** 2)

    i1 = _bin_codes(d1_ref[...], lowers_ref[0])
    i2 = _bin_codes(d2_ref[...], lowers_ref[1])
    i3 = _bin_codes(d3_ref[...], lowers_ref[2])
    t123 = (i1 * _NBINS + i2) * _NBINS + i3

    tT = t123.T

    row_iota = jax.lax.broadcasted_iota(jnp.int32, (_NBINS, D), 0)
    _dn_t = (((1,), (1,)), ((), ()))

    acc1 = jnp.zeros((_NBINS, 1), f32)
    acc2 = jnp.zeros((1, _NBINS), f32)
    acc3 = jnp.zeros((1, _NBINS), f32)
    acc12 = jnp.zeros((1, _NBINS), f32)
    acc13 = jnp.zeros((1, _NBINS), f32)
    acc23 = jnp.zeros((1, _NBINS), f32)
    acc123 = jnp.zeros((1, 128), f32)

    for r in range(R):
        ohT1 = jnp.where(row_iota == i1[r:r + 1, :], 1.0, 0.0).astype(jnp.bfloat16)
        ohT2 = jnp.where(row_iota == i2[r:r + 1, :], 1.0, 0.0).astype(jnp.bfloat16)
        ohT3 = jnp.where(row_iota == i3[r:r + 1, :], 1.0, 0.0).astype(jnp.bfloat16)

        h12 = jax.lax.dot_general(ohT1, ohT2, _dn_t,
                                  preferred_element_type=f32)
        h13 = jax.lax.dot_general(ohT1, ohT3, _dn_t,
                                  preferred_element_type=f32)
        h23 = jax.lax.dot_general(ohT2, ohT3, _dn_t,
                                  preferred_element_type=f32)

        acc12 = acc12 + jnp.sum(_xlogx(h12), axis=0, keepdims=True)
        acc13 = acc13 + jnp.sum(_xlogx(h13), axis=0, keepdims=True)
        acc23 = acc23 + jnp.sum(_xlogx(h23), axis=0, keepdims=True)
        acc1 = acc1 + _xlogx(jnp.sum(h12, axis=1, keepdims=True))
        acc2 = acc2 + _xlogx(jnp.sum(h12, axis=0, keepdims=True))
        acc3 = acc3 + _xlogx(jnp.sum(h13, axis=0, keepdims=True))

        tcol = tT[:, r:r + 1]
        for c in range(D // 128):
            trow_c = t123[r:r + 1, c * 128:(c + 1) * 128]
            eq = jnp.where(tcol == trow_c, 1.0, 0.0)
            cnt = jnp.sum(eq, axis=0, keepdims=True)
            acc123 = acc123 + jnp.log(cnt)

    s1 = jnp.sum(acc1)
    s2 = jnp.sum(acc2)
    s3 = jnp.sum(acc3)
    s12 = jnp.sum(acc12)
    s13 = jnp.sum(acc13)
    s23 = jnp.sum(acc23)
    s123 = jnp.sum(acc123)

    ds = (d1_ref[...], d2_ref[...], d3_ref[...])
    os_ = (o1_ref[...], o2_ref[...], o3_ref[...])
    m = [jnp.max(d, axis=1, keepdims=True) for d in ds]
    e = [jnp.exp(ds[a] - m[a]) for a in range(3)]
    s = [jnp.sum(e[a], axis=1, keepdims=True) for a in range(3)]
    U = [jnp.sum(e[a] * os_[a], axis=1, keepdims=True) for a in range(3)]
    V = [jnp.sum(e[a] * ds[a], axis=1, keepdims=True) for a in range(3)]
    mo = [jnp.max(o, axis=1, keepdims=True) for o in os_]
    so = [jnp.sum(jnp.exp(os_[a] - mo[a]), axis=1, keepdims=True) for a in range(3)]

    def _xc(C):
        mstar = m[C[0]]
        mostar = mo[C[0]]
        for a in C[1:]:
            mstar = jnp.maximum(mstar, m[a])
            mostar = jnp.maximum(mostar, mo[a])
        Z = sum(jnp.exp(m[a] - mstar) * s[a] for a in C)
        TO = sum(jnp.exp(m[a] - mstar) * U[a] for a in C) / Z
        TD = sum(jnp.exp(m[a] - mstar) * V[a] for a in C) / Z
        Zo = sum(jnp.exp(mo[a] - mostar) * so[a] for a in C)
        lse_d = mstar + jnp.log(Z)
        lse_o = mostar + jnp.log(Zo)
        P = TO - lse_o
        T = TD - lse_d
        nC = len(C) * D
        return jnp.sum(-P - (T - P) / nC)

    x1 = _xc((0,))
    x2 = _xc((1,))
    x3 = _xc((2,))
    x13 = _xc((0, 2))
    x23 = _xc((1, 2))
    x12 = _xc((0, 1))
    x123 = _xc((0, 1, 2))

    vals = (mse_p, s1, s2, s3, s12, s13, s23, s123,
            x1, x2, x3, x13, x23, x12, x123)
    lane = jax.lax.broadcasted_iota(jnp.int32, (1, 128), 1)
    out_v = jnp.zeros((1, 128), f32)
    for k, v in enumerate(vals):
        out_v = jnp.where(lane == k, v, out_v)
    o_ref[...] = out_v.reshape(1, 1, 128)


def kernel(data, data1, data2, data3, output1, output2, output3, output):
    B, D = data1.shape
    f32 = jnp.float32

    mb = min(_MIN_BLOCK, B)
    nb1 = B // mb
    mins = pl.pallas_call(
        _min_kernel,
        out_shape=jax.ShapeDtypeStruct((nb1, 1, 128), f32),
        grid=(nb1,),
        in_specs=[pl.BlockSpec((mb, D), lambda i: (i, 0))] * 3,
        out_specs=pl.BlockSpec((1, 1, 128), lambda i: (i, 0, 0)),
        compiler_params=pltpu.CompilerParams(
            dimension_semantics=("parallel",)),
        name="sae_loss_mins",
    )(data1, data2, data3)
    lowers = jnp.floor(jnp.min(mins[:, 0, :3], axis=0))

    R = _ROWS_PER_BLOCK
    nb2 = B // R
    part = pl.pallas_call(
        _main_kernel,
        out_shape=jax.ShapeDtypeStruct((nb2, 1, 128), f32),
        grid=(nb2,),
        in_specs=[
            pl.BlockSpec((R, D), lambda i: (i, 0)),
            pl.BlockSpec((R, D), lambda i: (i, 0)),
            pl.BlockSpec((R, D), lambda i: (i, 0)),
            pl.BlockSpec((R, D), lambda i: (i, 0)),
            pl.BlockSpec((R, D), lambda i: (i, 0)),
            pl.BlockSpec((R, D), lambda i: (i, 0)),
            pl.BlockSpec((R, 3 * D), lambda i: (i, 0)),
            pl.BlockSpec((R, 3 * D), lambda i: (i, 0)),
            pl.BlockSpec(memory_space=pltpu.SMEM),
        ],
        out_specs=pl.BlockSpec((1, 1, 128), lambda i: (i, 0, 0)),
        compiler_params=pltpu.CompilerParams(
            dimension_semantics=("parallel",),
            vmem_limit_bytes=56 * 1024 * 1024),
        name="sae_loss_main",
    )(data1, data2, data3, output1, output2, output3, data, output, lowers)

    sums = jnp.sum(part[:, 0, :], axis=0)
    (mse_s, s1, s2, s3, s12, s13, s23, s123,
     x1, x2, x3, x13, x23, x12, x123) = [sums[k] for k in range(15)]

    n = f32(D)
    logn = jnp.log(n)
    Bf = f32(B)

    mse = 0.5 * mse_s / (Bf * 3 * n)
    H_d1 = logn - s1 / (Bf * n)
    H_d2 = logn - s2 / (Bf * n)
    H_d3 = logn - s3 / (Bf * n)
    H_in13 = logn - s13 / (Bf * n)
    H_in23 = logn - s23 / (Bf * n)
    H_in12 = logn - s12 / (Bf * n)

    H_o1 = x1 / Bf
    H_o2 = x2 / Bf
    H_o3 = x3 / Bf
    H_o13 = x13 / Bf
    H_o23 = x23 / Bf
    H_o12 = x12 / Bf
    H_o123 = x123 / Bf

    H_1 = H_d1 - H_o1
    H_2 = H_d2 - H_o2
    H_3 = H_d3 - H_o3

    H_MI13 = (H_o1 + H_o3 - H_o13) - (H_d1 + H_d3 - H_in13)
    H_MI23 = (H_o2 + H_o3 - H_o23) - (H_d2 + H_d3 - H_in23)
    H_MI12 = (H_o1 + H_o2 - H_o12) - (H_d1 + H_d2 - H_in12)

    data_mu = (s3 + s123 - s13 - s23) / n
    label_cmi = H_o23 - H_o3 + H_o13 - H_o123
    CMI = label_cmi - data_mu

    return 0.9 * mse + 0.1 * (H_1 ** 2 + H_2 ** 2 + H_3 ** 2
                              + H_MI13 ** 2 + H_MI23 ** 2 + H_MI12 ** 2
                              + CMI ** 2)


# speedup vs baseline: 26.8924x; 1.0540x over previous
import jax
import jax.numpy as jnp
from jax.experimental import pallas as pl
from jax.experimental.pallas import tpu as pltpu

_STEP = 0.175
_NBINS = 128
_ROWS_PER_BLOCK = 8
_MIN_BLOCK = 512


def _min_kernel(d1_ref, d2_ref, d3_ref, o_ref):
    lane = jax.lax.broadcasted_iota(jnp.int32, (1, 128), 1)
    m1 = jnp.min(d1_ref[...])
    m2 = jnp.min(d2_ref[...])
    m3 = jnp.min(d3_ref[...])
    v = jnp.where(lane == 0, m1,
                  jnp.where(lane == 1, m2,
                            jnp.where(lane == 2, m3, jnp.float32(0))))
    o_ref[...] = v.reshape(1, 1, 128)


def _bin_codes(x, lower):
    idx = jnp.ceil((x - lower) / _STEP) - 1.0
    return jnp.clip(idx, 0.0, float(_NBINS - 1)).astype(jnp.int32)


def _xlogx(h):
    return h * jnp.log(jnp.maximum(h, 1.0))


def _main_kernel(d1_ref, d2_ref, d3_ref, o1_ref, o2_ref, o3_ref,
                 data_ref, outp_ref, lowers_ref, o_ref):
    D = d1_ref.shape[1]
    R = d1_ref.shape[0]
    f32 = jnp.float32

    mse_p = jnp.sum((data_ref[...] - outp_ref[...]) ** 2)

    i1 = _bin_codes(d1_ref[...], lowers_ref[0])
    i2 = _bin_codes(d2_ref[...], lowers_ref[1])
    i3 = _bin_codes(d3_ref[...], lowers_ref[2])
    t123 = (i1 * _NBINS + i2) * _NBINS + i3

    tT = t123.T

    bf16 = jnp.bfloat16
    row_iota = jax.lax.broadcasted_iota(jnp.int32, (_NBINS, D), 0).astype(bf16)
    i1b = i1.astype(bf16)
    i2b = i2.astype(bf16)
    i3b = i3.astype(bf16)
    one_b = jnp.array(1, bf16)
    zero_b = jnp.array(0, bf16)
    _dn_t = (((1,), (1,)), ((), ()))

    acc1 = jnp.zeros((1, _NBINS), f32)
    acc2 = jnp.zeros((1, _NBINS), f32)
    acc3 = jnp.zeros((1, _NBINS), f32)
    acc12 = jnp.zeros((1, _NBINS), f32)
    acc13 = jnp.zeros((1, _NBINS), f32)
    acc23 = jnp.zeros((1, _NBINS), f32)
    acc123 = jnp.zeros((1, 128), f32)

    for r in range(R):
        ohT1 = jnp.where(row_iota == i1b[r:r + 1, :], one_b, zero_b)
        ohT2 = jnp.where(row_iota == i2b[r:r + 1, :], one_b, zero_b)
        ohT3 = jnp.where(row_iota == i3b[r:r + 1, :], one_b, zero_b)

        h21 = jax.lax.dot_general(ohT2, ohT1, _dn_t,
                                  preferred_element_type=f32)
        h13 = jax.lax.dot_general(ohT1, ohT3, _dn_t,
                                  preferred_element_type=f32)
        h32 = jax.lax.dot_general(ohT3, ohT2, _dn_t,
                                  preferred_element_type=f32)

        acc12 = acc12 + jnp.sum(_xlogx(h21), axis=0, keepdims=True)
        acc13 = acc13 + jnp.sum(_xlogx(h13), axis=0, keepdims=True)
        acc23 = acc23 + jnp.sum(_xlogx(h32), axis=0, keepdims=True)
        acc1 = acc1 + _xlogx(jnp.sum(h21, axis=0, keepdims=True))
        acc2 = acc2 + _xlogx(jnp.sum(h32, axis=0, keepdims=True))
        acc3 = acc3 + _xlogx(jnp.sum(h13, axis=0, keepdims=True))

        tcol = tT[:, r:r + 1]
        for c in range(D // 128):
            trow_c = t123[r:r + 1, c * 128:(c + 1) * 128]
            eq = jnp.where(tcol == trow_c, 1.0, 0.0)
            cnt = jnp.sum(eq, axis=0, keepdims=True)
            acc123 = acc123 + jnp.log(cnt)

    s1 = jnp.sum(acc1)
    s2 = jnp.sum(acc2)
    s3 = jnp.sum(acc3)
    s12 = jnp.sum(acc12)
    s13 = jnp.sum(acc13)
    s23 = jnp.sum(acc23)
    s123 = jnp.sum(acc123)

    ds = (d1_ref[...], d2_ref[...], d3_ref[...])
    os_ = (o1_ref[...], o2_ref[...], o3_ref[...])
    m = [jnp.max(d, axis=1, keepdims=True) for d in ds]
    e = [jnp.exp(ds[a] - m[a]) for a in range(3)]
    s = [jnp.sum(e[a], axis=1, keepdims=True) for a in range(3)]
    U = [jnp.sum(e[a] * os_[a], axis=1, keepdims=True) for a in range(3)]
    V = [jnp.sum(e[a] * ds[a], axis=1, keepdims=True) for a in range(3)]
    mo = [jnp.max(o, axis=1, keepdims=True) for o in os_]
    so = [jnp.sum(jnp.exp(os_[a] - mo[a]), axis=1, keepdims=True) for a in range(3)]

    def _xc(C):
        mstar = m[C[0]]
        mostar = mo[C[0]]
        for a in C[1:]:
            mstar = jnp.maximum(mstar, m[a])
            mostar = jnp.maximum(mostar, mo[a])
        Z = sum(jnp.exp(m[a] - mstar) * s[a] for a in C)
        TO = sum(jnp.exp(m[a] - mstar) * U[a] for a in C) / Z
        TD = sum(jnp.exp(m[a] - mstar) * V[a] for a in C) / Z
        Zo = sum(jnp.exp(mo[a] - mostar) * so[a] for a in C)
        lse_d = mstar + jnp.log(Z)
        lse_o = mostar + jnp.log(Zo)
        P = TO - lse_o
        T = TD - lse_d
        nC = len(C) * D
        return jnp.sum(-P - (T - P) / nC)

    x1 = _xc((0,))
    x2 = _xc((1,))
    x3 = _xc((2,))
    x13 = _xc((0, 2))
    x23 = _xc((1, 2))
    x12 = _xc((0, 1))
    x123 = _xc((0, 1, 2))

    vals = (mse_p, s1, s2, s3, s12, s13, s23, s123,
            x1, x2, x3, x13, x23, x12, x123)
    lane = jax.lax.broadcasted_iota(jnp.int32, (1, 128), 1)
    out_v = jnp.zeros((1, 128), f32)
    for k, v in enumerate(vals):
        out_v = jnp.where(lane == k, v, out_v)
    o_ref[...] = out_v.reshape(1, 1, 128)


def kernel(data, data1, data2, data3, output1, output2, output3, output):
    B, D = data1.shape
    f32 = jnp.float32

    mb = min(_MIN_BLOCK, B)
    nb1 = B // mb
    mins = pl.pallas_call(
        _min_kernel,
        out_shape=jax.ShapeDtypeStruct((nb1, 1, 128), f32),
        grid=(nb1,),
        in_specs=[pl.BlockSpec((mb, D), lambda i: (i, 0))] * 3,
        out_specs=pl.BlockSpec((1, 1, 128), lambda i: (i, 0, 0)),
        compiler_params=pltpu.CompilerParams(
            dimension_semantics=("parallel",)),
        name="sae_loss_mins",
    )(data1, data2, data3)
    lowers = jnp.floor(jnp.min(mins[:, 0, :3], axis=0))

    R = _ROWS_PER_BLOCK
    nb2 = B // R
    part = pl.pallas_call(
        _main_kernel,
        out_shape=jax.ShapeDtypeStruct((nb2, 1, 128), f32),
        grid=(nb2,),
        in_specs=[
            pl.BlockSpec((R, D), lambda i: (i, 0)),
            pl.BlockSpec((R, D), lambda i: (i, 0)),
            pl.BlockSpec((R, D), lambda i: (i, 0)),
            pl.BlockSpec((R, D), lambda i: (i, 0)),
            pl.BlockSpec((R, D), lambda i: (i, 0)),
            pl.BlockSpec((R, D), lambda i: (i, 0)),
            pl.BlockSpec((R, 3 * D), lambda i: (i, 0)),
            pl.BlockSpec((R, 3 * D), lambda i: (i, 0)),
            pl.BlockSpec(memory_space=pltpu.SMEM),
        ],
        out_specs=pl.BlockSpec((1, 1, 128), lambda i: (i, 0, 0)),
        compiler_params=pltpu.CompilerParams(
            dimension_semantics=("parallel",),
            vmem_limit_bytes=56 * 1024 * 1024),
        name="sae_loss_main",
    )(data1, data2, data3, output1, output2, output3, data, output, lowers)

    sums = jnp.sum(part[:, 0, :], axis=0)
    (mse_s, s1, s2, s3, s12, s13, s23, s123,
     x1, x2, x3, x13, x23, x12, x123) = [sums[k] for k in range(15)]

    n = f32(D)
    logn = jnp.log(n)
    Bf = f32(B)

    mse = 0.5 * mse_s / (Bf * 3 * n)
    H_d1 = logn - s1 / (Bf * n)
    H_d2 = logn - s2 / (Bf * n)
    H_d3 = logn - s3 / (Bf * n)
    H_in13 = logn - s13 / (Bf * n)
    H_in23 = logn - s23 / (Bf * n)
    H_in12 = logn - s12 / (Bf * n)

    H_o1 = x1 / Bf
    H_o2 = x2 / Bf
    H_o3 = x3 / Bf
    H_o13 = x13 / Bf
    H_o23 = x23 / Bf
    H_o12 = x12 / Bf
    H_o123 = x123 / Bf

    H_1 = H_d1 - H_o1
    H_2 = H_d2 - H_o2
    H_3 = H_d3 - H_o3

    H_MI13 = (H_o1 + H_o3 - H_o13) - (H_d1 + H_d3 - H_in13)
    H_MI23 = (H_o2 + H_o3 - H_o23) - (H_d2 + H_d3 - H_in23)
    H_MI12 = (H_o1 + H_o2 - H_o12) - (H_d1 + H_d2 - H_in12)

    data_mu = (s3 + s123 - s13 - s23) / n
    label_cmi = H_o23 - H_o3 + H_o13 - H_o123
    CMI = label_cmi - data_mu

    return 0.9 * mse + 0.1 * (H_1 ** 2 + H_2 ** 2 + H_3 ** 2
                              + H_MI13 ** 2 + H_MI23 ** 2 + H_MI12 ** 2
                              + CMI ** 2)


# speedup vs baseline: 28.2911x; 1.0520x over previous
import jax
import jax.numpy as jnp
from jax.experimental import pallas as pl
from jax.experimental.pallas import tpu as pltpu

_STEP = 0.175
_NBINS = 128
_ROWS_PER_BLOCK = 16
_MIN_BLOCK = 512


def _min_kernel(d1_ref, d2_ref, d3_ref, o_ref):
    lane = jax.lax.broadcasted_iota(jnp.int32, (1, 128), 1)
    m1 = jnp.min(d1_ref[...])
    m2 = jnp.min(d2_ref[...])
    m3 = jnp.min(d3_ref[...])
    v = jnp.where(lane == 0, m1,
                  jnp.where(lane == 1, m2,
                            jnp.where(lane == 2, m3, jnp.float32(0))))
    o_ref[...] = v.reshape(1, 1, 128)


def _bin_codes(x, lower):
    idx = jnp.ceil((x - lower) / _STEP) - 1.0
    return jnp.clip(idx, 0.0, float(_NBINS - 1)).astype(jnp.int32)


def _xlogx(h):
    return h * jnp.log(jnp.maximum(h, 1.0))


def _main_kernel(d1_ref, d2_ref, d3_ref, o1_ref, o2_ref, o3_ref,
                 data_ref, outp_ref, lowers_ref, o_ref):
    D = d1_ref.shape[1]
    R = d1_ref.shape[0]
    f32 = jnp.float32

    mse_p = jnp.sum((data_ref[...] - outp_ref[...]) ** 2)

    i1 = _bin_codes(d1_ref[...], lowers_ref[0])
    i2 = _bin_codes(d2_ref[...], lowers_ref[1])
    i3 = _bin_codes(d3_ref[...], lowers_ref[2])
    t123 = (i1 * _NBINS + i2) * _NBINS + i3

    tT = t123.T

    bf16 = jnp.bfloat16
    row_iota = jax.lax.broadcasted_iota(jnp.int32, (_NBINS, D), 0).astype(bf16)
    i1b = i1.astype(bf16)
    i2b = i2.astype(bf16)
    i3b = i3.astype(bf16)
    one_b = jnp.array(1, bf16)
    zero_b = jnp.array(0, bf16)
    _dn_t = (((1,), (1,)), ((), ()))

    acc1 = jnp.zeros((1, _NBINS), f32)
    acc2 = jnp.zeros((1, _NBINS), f32)
    acc3 = jnp.zeros((1, _NBINS), f32)
    acc12 = jnp.zeros((1, _NBINS), f32)
    acc13 = jnp.zeros((1, _NBINS), f32)
    acc23 = jnp.zeros((1, _NBINS), f32)
    acc123 = jnp.zeros((1, 128), f32)

    for r in range(R):
        ohT1 = jnp.where(row_iota == i1b[r:r + 1, :], one_b, zero_b)
        ohT2 = jnp.where(row_iota == i2b[r:r + 1, :], one_b, zero_b)
        ohT3 = jnp.where(row_iota == i3b[r:r + 1, :], one_b, zero_b)

        h21 = jax.lax.dot_general(ohT2, ohT1, _dn_t,
                                  preferred_element_type=f32)
        h13 = jax.lax.dot_general(ohT1, ohT3, _dn_t,
                                  preferred_element_type=f32)
        h32 = jax.lax.dot_general(ohT3, ohT2, _dn_t,
                                  preferred_element_type=f32)

        acc12 = acc12 + jnp.sum(_xlogx(h21), axis=0, keepdims=True)
        acc13 = acc13 + jnp.sum(_xlogx(h13), axis=0, keepdims=True)
        acc23 = acc23 + jnp.sum(_xlogx(h32), axis=0, keepdims=True)
        acc1 = acc1 + _xlogx(jnp.sum(h21, axis=0, keepdims=True))
        acc2 = acc2 + _xlogx(jnp.sum(h32, axis=0, keepdims=True))
        acc3 = acc3 + _xlogx(jnp.sum(h13, axis=0, keepdims=True))

        tcol = tT[:, r:r + 1]
        for c in range(D // 128):
            trow_c = t123[r:r + 1, c * 128:(c + 1) * 128]
            eq = jnp.where(tcol == trow_c, 1.0, 0.0)
            cnt = jnp.sum(eq, axis=0, keepdims=True)
            acc123 = acc123 + jnp.log(cnt)

    s1 = jnp.sum(acc1)
    s2 = jnp.sum(acc2)
    s3 = jnp.sum(acc3)
    s12 = jnp.sum(acc12)
    s13 = jnp.sum(acc13)
    s23 = jnp.sum(acc23)
    s123 = jnp.sum(acc123)

    ds = (d1_ref[...], d2_ref[...], d3_ref[...])
    os_ = (o1_ref[...], o2_ref[...], o3_ref[...])
    m = [jnp.max(d, axis=1, keepdims=True) for d in ds]
    e = [jnp.exp(ds[a] - m[a]) for a in range(3)]
    s = [jnp.sum(e[a], axis=1, keepdims=True) for a in range(3)]
    U = [jnp.sum(e[a] * os_[a], axis=1, keepdims=True) for a in range(3)]
    V = [jnp.sum(e[a] * ds[a], axis=1, keepdims=True) for a in range(3)]
    mo = [jnp.max(o, axis=1, keepdims=True) for o in os_]
    so = [jnp.sum(jnp.exp(os_[a] - mo[a]), axis=1, keepdims=True) for a in range(3)]

    def _xc(C):
        mstar = m[C[0]]
        mostar = mo[C[0]]
        for a in C[1:]:
            mstar = jnp.maximum(mstar, m[a])
            mostar = jnp.maximum(mostar, mo[a])
        Z = sum(jnp.exp(m[a] - mstar) * s[a] for a in C)
        TO = sum(jnp.exp(m[a] - mstar) * U[a] for a in C) / Z
        TD = sum(jnp.exp(m[a] - mstar) * V[a] for a in C) / Z
        Zo = sum(jnp.exp(mo[a] - mostar) * so[a] for a in C)
        lse_d = mstar + jnp.log(Z)
        lse_o = mostar + jnp.log(Zo)
        P = TO - lse_o
        T = TD - lse_d
        nC = len(C) * D
        return jnp.sum(-P - (T - P) / nC)

    x1 = _xc((0,))
    x2 = _xc((1,))
    x3 = _xc((2,))
    x13 = _xc((0, 2))
    x23 = _xc((1, 2))
    x12 = _xc((0, 1))
    x123 = _xc((0, 1, 2))

    vals = (mse_p, s1, s2, s3, s12, s13, s23, s123,
            x1, x2, x3, x13, x23, x12, x123)
    lane = jax.lax.broadcasted_iota(jnp.int32, (1, 128), 1)
    out_v = jnp.zeros((1, 128), f32)
    for k, v in enumerate(vals):
        out_v = jnp.where(lane == k, v, out_v)
    o_ref[...] = out_v.reshape(1, 1, 128)


def kernel(data, data1, data2, data3, output1, output2, output3, output):
    B, D = data1.shape
    f32 = jnp.float32

    mb = min(_MIN_BLOCK, B)
    nb1 = B // mb
    mins = pl.pallas_call(
        _min_kernel,
        out_shape=jax.ShapeDtypeStruct((nb1, 1, 128), f32),
        grid=(nb1,),
        in_specs=[pl.BlockSpec((mb, D), lambda i: (i, 0))] * 3,
        out_specs=pl.BlockSpec((1, 1, 128), lambda i: (i, 0, 0)),
        compiler_params=pltpu.CompilerParams(
            dimension_semantics=("parallel",)),
        name="sae_loss_mins",
    )(data1, data2, data3)
    lowers = jnp.floor(jnp.min(mins[:, 0, :3], axis=0))

    R = _ROWS_PER_BLOCK
    nb2 = B // R
    part = pl.pallas_call(
        _main_kernel,
        out_shape=jax.ShapeDtypeStruct((nb2, 1, 128), f32),
        grid=(nb2,),
        in_specs=[
            pl.BlockSpec((R, D), lambda i: (i, 0)),
            pl.BlockSpec((R, D), lambda i: (i, 0)),
            pl.BlockSpec((R, D), lambda i: (i, 0)),
            pl.BlockSpec((R, D), lambda i: (i, 0)),
            pl.BlockSpec((R, D), lambda i: (i, 0)),
            pl.BlockSpec((R, D), lambda i: (i, 0)),
            pl.BlockSpec((R, 3 * D), lambda i: (i, 0)),
            pl.BlockSpec((R, 3 * D), lambda i: (i, 0)),
            pl.BlockSpec(memory_space=pltpu.SMEM),
        ],
        out_specs=pl.BlockSpec((1, 1, 128), lambda i: (i, 0, 0)),
        compiler_params=pltpu.CompilerParams(
            dimension_semantics=("parallel",),
            vmem_limit_bytes=56 * 1024 * 1024),
        name="sae_loss_main",
    )(data1, data2, data3, output1, output2, output3, data, output, lowers)

    sums = jnp.sum(part[:, 0, :], axis=0)
    (mse_s, s1, s2, s3, s12, s13, s23, s123,
     x1, x2, x3, x13, x23, x12, x123) = [sums[k] for k in range(15)]

    n = f32(D)
    logn = jnp.log(n)
    Bf = f32(B)

    mse = 0.5 * mse_s / (Bf * 3 * n)
    H_d1 = logn - s1 / (Bf * n)
    H_d2 = logn - s2 / (Bf * n)
    H_d3 = logn - s3 / (Bf * n)
    H_in13 = logn - s13 / (Bf * n)
    H_in23 = logn - s23 / (Bf * n)
    H_in12 = logn - s12 / (Bf * n)

    H_o1 = x1 / Bf
    H_o2 = x2 / Bf
    H_o3 = x3 / Bf
    H_o13 = x13 / Bf
    H_o23 = x23 / Bf
    H_o12 = x12 / Bf
    H_o123 = x123 / Bf

    H_1 = H_d1 - H_o1
    H_2 = H_d2 - H_o2
    H_3 = H_d3 - H_o3

    H_MI13 = (H_o1 + H_o3 - H_o13) - (H_d1 + H_d3 - H_in13)
    H_MI23 = (H_o2 + H_o3 - H_o23) - (H_d2 + H_d3 - H_in23)
    H_MI12 = (H_o1 + H_o2 - H_o12) - (H_d1 + H_d2 - H_in12)

    data_mu = (s3 + s123 - s13 - s23) / n
    label_cmi = H_o23 - H_o3 + H_o13 - H_o123
    CMI = label_cmi - data_mu

    return 0.9 * mse + 0.1 * (H_1 ** 2 + H_2 ** 2 + H_3 ** 2
                              + H_MI13 ** 2 + H_MI23 ** 2 + H_MI12 ** 2
                              + CMI ** 2)


# speedup vs baseline: 50.7192x; 1.7928x over previous
import jax
import jax.numpy as jnp
from jax.experimental import pallas as pl
from jax.experimental.pallas import tpu as pltpu

_STEP = 0.175
_NBINS = 128
_ROWS_PER_BLOCK = 16
_MIN_BLOCK = 512


def _min_kernel(d1_ref, d2_ref, d3_ref, o_ref):
    lane = jax.lax.broadcasted_iota(jnp.int32, (1, 128), 1)
    m1 = jnp.min(d1_ref[...])
    m2 = jnp.min(d2_ref[...])
    m3 = jnp.min(d3_ref[...])
    v = jnp.where(lane == 0, m1,
                  jnp.where(lane == 1, m2,
                            jnp.where(lane == 2, m3, jnp.float32(0))))
    o_ref[...] = v.reshape(1, 1, 128)


def _bin_codes(x, lower):
    idx = jnp.ceil((x - lower) / _STEP) - 1.0
    return jnp.clip(idx, 0.0, float(_NBINS - 1)).astype(jnp.int32)


def _xlogx(h):
    return h * jnp.log(jnp.maximum(h, 1.0))


def _roll_lanes(x, j):
    return jnp.concatenate([x[:, j:], x[:, :j]], axis=1)


def _bitonic_sort_lanes(x, lane_idx):
    R, D = x.shape
    k = 2
    kl = 1
    while k <= D:
        j = k // 2
        jl = kl - 1
        while j >= 1:
            a = _roll_lanes(x, j)
            b = _roll_lanes(x, D - j)
            low = (lane_idx & j) == 0
            samebit = ((jax.lax.shift_right_logical(lane_idx, jl)
                        ^ jax.lax.shift_right_logical(lane_idx, kl)) & 1) == 0
            partner = jnp.where(low, a, b)
            mn = jnp.minimum(x, partner)
            mx = jnp.maximum(x, partner)
            x = jnp.where(samebit, mn, mx)
            j //= 2
            jl -= 1
        k *= 2
        kl += 1
    return x


def _main_kernel(d1_ref, d2_ref, d3_ref, o1_ref, o2_ref, o3_ref,
                 data_ref, outp_ref, lowers_ref, o_ref):
    D = d1_ref.shape[1]
    R = d1_ref.shape[0]
    f32 = jnp.float32

    mse_p = jnp.sum((data_ref[...] - outp_ref[...]) ** 2)

    i1 = _bin_codes(d1_ref[...], lowers_ref[0])
    i2 = _bin_codes(d2_ref[...], lowers_ref[1])
    i3 = _bin_codes(d3_ref[...], lowers_ref[2])
    t123 = (i1 * _NBINS + i2) * _NBINS + i3

    bf16 = jnp.bfloat16
    row_iota = jax.lax.broadcasted_iota(jnp.int32, (_NBINS, D), 0).astype(bf16)
    i1b = i1.astype(bf16)
    i2b = i2.astype(bf16)
    i3b = i3.astype(bf16)
    one_b = jnp.array(1, bf16)
    zero_b = jnp.array(0, bf16)
    _dn_t = (((1,), (1,)), ((), ()))

    acc1 = jnp.zeros((1, _NBINS), f32)
    acc2 = jnp.zeros((1, _NBINS), f32)
    acc3 = jnp.zeros((1, _NBINS), f32)
    acc12 = jnp.zeros((1, _NBINS), f32)
    acc13 = jnp.zeros((1, _NBINS), f32)
    acc23 = jnp.zeros((1, _NBINS), f32)

    for r in range(R):
        ohT1 = jnp.where(row_iota == i1b[r:r + 1, :], one_b, zero_b)
        ohT2 = jnp.where(row_iota == i2b[r:r + 1, :], one_b, zero_b)
        ohT3 = jnp.where(row_iota == i3b[r:r + 1, :], one_b, zero_b)

        h21 = jax.lax.dot_general(ohT2, ohT1, _dn_t,
                                  preferred_element_type=f32)
        h13 = jax.lax.dot_general(ohT1, ohT3, _dn_t,
                                  preferred_element_type=f32)
        h32 = jax.lax.dot_general(ohT3, ohT2, _dn_t,
                                  preferred_element_type=f32)

        acc12 = acc12 + jnp.sum(_xlogx(h21), axis=0, keepdims=True)
        acc13 = acc13 + jnp.sum(_xlogx(h13), axis=0, keepdims=True)
        acc23 = acc23 + jnp.sum(_xlogx(h32), axis=0, keepdims=True)
        acc1 = acc1 + _xlogx(jnp.sum(h21, axis=0, keepdims=True))
        acc2 = acc2 + _xlogx(jnp.sum(h32, axis=0, keepdims=True))
        acc3 = acc3 + _xlogx(jnp.sum(h13, axis=0, keepdims=True))


    lane_idx = jax.lax.broadcasted_iota(jnp.int32, (R, D), 1)
    srt = _bitonic_sort_lanes(t123, lane_idx)
    prev = _roll_lanes(srt, D - 1)
    nxt = _roll_lanes(srt, 1)
    start = jnp.where(srt != prev, lane_idx, 0)
    end = jnp.where(srt != nxt, lane_idx, D + 1)
    end = jnp.where(lane_idx == D - 1, jnp.int32(D - 1), end)
    k = 1
    while k < D:
        start = jnp.maximum(
            start, jnp.concatenate([jnp.zeros((R, k), jnp.int32),
                                    start[:, :-k]], axis=1))
        end = jnp.minimum(
            end, jnp.concatenate([end[:, k:],
                                  jnp.full((R, k), D + 1, jnp.int32)], axis=1))
        k *= 2
    cntf = (end - start + 1).astype(f32)
    s123 = jnp.sum(jnp.log(cntf))

    s1 = jnp.sum(acc1)
    s2 = jnp.sum(acc2)
    s3 = jnp.sum(acc3)
    s12 = jnp.sum(acc12)
    s13 = jnp.sum(acc13)
    s23 = jnp.sum(acc23)

    ds = (d1_ref[...], d2_ref[...], d3_ref[...])
    os_ = (o1_ref[...], o2_ref[...], o3_ref[...])
    m = [jnp.max(d, axis=1, keepdims=True) for d in ds]
    e = [jnp.exp(ds[a] - m[a]) for a in range(3)]
    s = [jnp.sum(e[a], axis=1, keepdims=True) for a in range(3)]
    U = [jnp.sum(e[a] * os_[a], axis=1, keepdims=True) for a in range(3)]
    V = [jnp.sum(e[a] * ds[a], axis=1, keepdims=True) for a in range(3)]
    mo = [jnp.max(o, axis=1, keepdims=True) for o in os_]
    so = [jnp.sum(jnp.exp(os_[a] - mo[a]), axis=1, keepdims=True) for a in range(3)]

    def _xc(C):
        mstar = m[C[0]]
        mostar = mo[C[0]]
        for a in C[1:]:
            mstar = jnp.maximum(mstar, m[a])
            mostar = jnp.maximum(mostar, mo[a])
        Z = sum(jnp.exp(m[a] - mstar) * s[a] for a in C)
        TO = sum(jnp.exp(m[a] - mstar) * U[a] for a in C) / Z
        TD = sum(jnp.exp(m[a] - mstar) * V[a] for a in C) / Z
        Zo = sum(jnp.exp(mo[a] - mostar) * so[a] for a in C)
        lse_d = mstar + jnp.log(Z)
        lse_o = mostar + jnp.log(Zo)
        P = TO - lse_o
        T = TD - lse_d
        nC = len(C) * D
        return jnp.sum(-P - (T - P) / nC)

    x1 = _xc((0,))
    x2 = _xc((1,))
    x3 = _xc((2,))
    x13 = _xc((0, 2))
    x23 = _xc((1, 2))
    x12 = _xc((0, 1))
    x123 = _xc((0, 1, 2))

    vals = (mse_p, s1, s2, s3, s12, s13, s23, s123,
            x1, x2, x3, x13, x23, x12, x123)
    lane = jax.lax.broadcasted_iota(jnp.int32, (1, 128), 1)
    out_v = jnp.zeros((1, 128), f32)
    for k, v in enumerate(vals):
        out_v = jnp.where(lane == k, v, out_v)
    o_ref[...] = out_v.reshape(1, 1, 128)


def kernel(data, data1, data2, data3, output1, output2, output3, output):
    B, D = data1.shape
    f32 = jnp.float32

    mb = min(_MIN_BLOCK, B)
    nb1 = B // mb
    mins = pl.pallas_call(
        _min_kernel,
        out_shape=jax.ShapeDtypeStruct((nb1, 1, 128), f32),
        grid=(nb1,),
        in_specs=[pl.BlockSpec((mb, D), lambda i: (i, 0))] * 3,
        out_specs=pl.BlockSpec((1, 1, 128), lambda i: (i, 0, 0)),
        compiler_params=pltpu.CompilerParams(
            dimension_semantics=("parallel",)),
        name="sae_loss_mins",
    )(data1, data2, data3)
    lowers = jnp.floor(jnp.min(mins[:, 0, :3], axis=0))

    R = _ROWS_PER_BLOCK
    nb2 = B // R
    part = pl.pallas_call(
        _main_kernel,
        out_shape=jax.ShapeDtypeStruct((nb2, 1, 128), f32),
        grid=(nb2,),
        in_specs=[
            pl.BlockSpec((R, D), lambda i: (i, 0)),
            pl.BlockSpec((R, D), lambda i: (i, 0)),
            pl.BlockSpec((R, D), lambda i: (i, 0)),
            pl.BlockSpec((R, D), lambda i: (i, 0)),
            pl.BlockSpec((R, D), lambda i: (i, 0)),
            pl.BlockSpec((R, D), lambda i: (i, 0)),
            pl.BlockSpec((R, 3 * D), lambda i: (i, 0)),
            pl.BlockSpec((R, 3 * D), lambda i: (i, 0)),
            pl.BlockSpec(memory_space=pltpu.SMEM),
        ],
        out_specs=pl.BlockSpec((1, 1, 128), lambda i: (i, 0, 0)),
        compiler_params=pltpu.CompilerParams(
            dimension_semantics=("parallel",),
            vmem_limit_bytes=56 * 1024 * 1024),
        name="sae_loss_main",
    )(data1, data2, data3, output1, output2, output3, data, output, lowers)

    sums = jnp.sum(part[:, 0, :], axis=0)
    (mse_s, s1, s2, s3, s12, s13, s23, s123,
     x1, x2, x3, x13, x23, x12, x123) = [sums[k] for k in range(15)]

    n = f32(D)
    logn = jnp.log(n)
    Bf = f32(B)

    mse = 0.5 * mse_s / (Bf * 3 * n)
    H_d1 = logn - s1 / (Bf * n)
    H_d2 = logn - s2 / (Bf * n)
    H_d3 = logn - s3 / (Bf * n)
    H_in13 = logn - s13 / (Bf * n)
    H_in23 = logn - s23 / (Bf * n)
    H_in12 = logn - s12 / (Bf * n)

    H_o1 = x1 / Bf
    H_o2 = x2 / Bf
    H_o3 = x3 / Bf
    H_o13 = x13 / Bf
    H_o23 = x23 / Bf
    H_o12 = x12 / Bf
    H_o123 = x123 / Bf

    H_1 = H_d1 - H_o1
    H_2 = H_d2 - H_o2
    H_3 = H_d3 - H_o3

    H_MI13 = (H_o1 + H_o3 - H_o13) - (H_d1 + H_d3 - H_in13)
    H_MI23 = (H_o2 + H_o3 - H_o23) - (H_d2 + H_d3 - H_in23)
    H_MI12 = (H_o1 + H_o2 - H_o12) - (H_d1 + H_d2 - H_in12)

    data_mu = (s3 + s123 - s13 - s23) / n
    label_cmi = H_o23 - H_o3 + H_o13 - H_o123
    CMI = label_cmi - data_mu

    return 0.9 * mse + 0.1 * (H_1 ** 2 + H_2 ** 2 + H_3 ** 2
                              + H_MI13 ** 2 + H_MI23 ** 2 + H_MI12 ** 2
                              + CMI ** 2)


# speedup vs baseline: 51.9193x; 1.0237x over previous
import jax
import jax.numpy as jnp
from jax.experimental import pallas as pl
from jax.experimental.pallas import tpu as pltpu

_STEP = 0.175
_NBINS = 128
_ROWS_PER_BLOCK = 16
_MIN_BLOCK = 512


def _min_kernel(d1_ref, d2_ref, d3_ref, o_ref):
    lane = jax.lax.broadcasted_iota(jnp.int32, (1, 128), 1)
    m1 = jnp.min(d1_ref[...])
    m2 = jnp.min(d2_ref[...])
    m3 = jnp.min(d3_ref[...])
    v = jnp.where(lane == 0, m1,
                  jnp.where(lane == 1, m2,
                            jnp.where(lane == 2, m3, jnp.float32(0))))
    o_ref[...] = v.reshape(1, 1, 128)


def _bin_codes(x, lower):
    idx = jnp.ceil((x - lower) / _STEP) - 1.0
    return jnp.clip(idx, 0.0, float(_NBINS - 1)).astype(jnp.int32)


def _xlogx(h):
    return h * jnp.log(jnp.maximum(h, 1.0))


def _roll_lanes(x, j):
    return jnp.concatenate([x[:, j:], x[:, :j]], axis=1)


def _bitonic_sort_lanes(x, lane_idx):
    R, D = x.shape
    nbits = D.bit_length() - 1
    bit = [jax.lax.shift_right_logical(lane_idx, n) & 1 for n in range(nbits)]
    k = 2
    kl = 1
    while k <= D:
        j = k // 2
        jl = kl - 1
        while j >= 1:
            a = _roll_lanes(x, j)
            b = _roll_lanes(x, D - j)
            low = bit[jl] == 0
            samebit = low if k == D else (bit[jl] == bit[kl])
            partner = jnp.where(low, a, b)
            mn = jnp.minimum(x, partner)
            mx = jnp.maximum(x, partner)
            x = jnp.where(samebit, mn, mx)
            j //= 2
            jl -= 1
        k *= 2
        kl += 1
    return x


def _runs_logsum(srt, lane_idx, D):
    f32 = jnp.float32
    R = srt.shape[0]
    prev = _roll_lanes(srt, D - 1)
    nxt = _roll_lanes(srt, 1)
    start = jnp.where(srt != prev, lane_idx, 0)
    end = jnp.where(srt != nxt, lane_idx, D + 1)
    end = jnp.where(lane_idx == D - 1, jnp.int32(D - 1), end)
    k = 1
    while k < D:
        start = jnp.maximum(
            start, jnp.concatenate([jnp.zeros((R, k), jnp.int32),
                                    start[:, :-k]], axis=1))
        end = jnp.minimum(
            end, jnp.concatenate([end[:, k:],
                                  jnp.full((R, k), D + 1, jnp.int32)], axis=1))
        k *= 2
    cntf = (end - start + 1).astype(f32)
    return jnp.sum(jnp.log(cntf))


def _main_kernel(d1_ref, d2_ref, d3_ref, o1_ref, o2_ref, o3_ref,
                 data_ref, outp_ref, lowers_ref, o_ref):
    D = d1_ref.shape[1]
    R = d1_ref.shape[0]
    f32 = jnp.float32

    mse_p = jnp.sum((data_ref[...] - outp_ref[...]) ** 2)

    i1 = _bin_codes(d1_ref[...], lowers_ref[0])
    i2 = _bin_codes(d2_ref[...], lowers_ref[1])
    i3 = _bin_codes(d3_ref[...], lowers_ref[2])
    t123 = (i1 * _NBINS + i2) * _NBINS + i3

    bf16 = jnp.bfloat16
    row_iota = jax.lax.broadcasted_iota(jnp.int32, (_NBINS, D), 0).astype(bf16)
    i1b = i1.astype(bf16)
    i2b = i2.astype(bf16)
    i3b = i3.astype(bf16)
    one_b = jnp.array(1, bf16)
    zero_b = jnp.array(0, bf16)
    _dn_t = (((1,), (1,)), ((), ()))

    acc1 = jnp.zeros((1, _NBINS), f32)
    acc12 = jnp.zeros((1, _NBINS), f32)
    acc2 = jnp.zeros((1, _NBINS), f32)
    acc3 = jnp.zeros((1, _NBINS), f32)
    acc13 = jnp.zeros((1, _NBINS), f32)
    acc23 = jnp.zeros((1, _NBINS), f32)

    for r in range(R):
        ohT1 = jnp.where(row_iota == i1b[r:r + 1, :], one_b, zero_b)
        ohT2 = jnp.where(row_iota == i2b[r:r + 1, :], one_b, zero_b)
        ohT3 = jnp.where(row_iota == i3b[r:r + 1, :], one_b, zero_b)

        h21 = jax.lax.dot_general(ohT2, ohT1, _dn_t,
                                  preferred_element_type=f32)
        h13 = jax.lax.dot_general(ohT1, ohT3, _dn_t,
                                  preferred_element_type=f32)
        h32 = jax.lax.dot_general(ohT3, ohT2, _dn_t,
                                  preferred_element_type=f32)

        acc12 = acc12 + jnp.sum(_xlogx(h21), axis=0, keepdims=True)
        acc13 = acc13 + jnp.sum(_xlogx(h13), axis=0, keepdims=True)
        acc23 = acc23 + jnp.sum(_xlogx(h32), axis=0, keepdims=True)
        acc1 = acc1 + _xlogx(jnp.sum(h21, axis=0, keepdims=True))
        acc2 = acc2 + _xlogx(jnp.sum(h32, axis=0, keepdims=True))
        acc3 = acc3 + _xlogx(jnp.sum(h13, axis=0, keepdims=True))

    lane_idx = jax.lax.broadcasted_iota(jnp.int32, (R, D), 1)
    srt = _bitonic_sort_lanes(t123, lane_idx)
    s123 = _runs_logsum(srt, lane_idx, D)

    s1 = jnp.sum(acc1)
    s2 = jnp.sum(acc2)
    s3 = jnp.sum(acc3)
    s12 = jnp.sum(acc12)
    s13 = jnp.sum(acc13)
    s23 = jnp.sum(acc23)

    ds = (d1_ref[...], d2_ref[...], d3_ref[...])
    os_ = (o1_ref[...], o2_ref[...], o3_ref[...])
    m = [jnp.max(d, axis=1, keepdims=True) for d in ds]
    e = [jnp.exp(ds[a] - m[a]) for a in range(3)]
    s = [jnp.sum(e[a], axis=1, keepdims=True) for a in range(3)]
    U = [jnp.sum(e[a] * os_[a], axis=1, keepdims=True) for a in range(3)]
    V = [jnp.sum(e[a] * ds[a], axis=1, keepdims=True) for a in range(3)]
    mo = [jnp.max(o, axis=1, keepdims=True) for o in os_]
    so = [jnp.sum(jnp.exp(os_[a] - mo[a]), axis=1, keepdims=True) for a in range(3)]

    def _xc(C):
        mstar = m[C[0]]
        mostar = mo[C[0]]
        for a in C[1:]:
            mstar = jnp.maximum(mstar, m[a])
            mostar = jnp.maximum(mostar, mo[a])
        Z = sum(jnp.exp(m[a] - mstar) * s[a] for a in C)
        TO = sum(jnp.exp(m[a] - mstar) * U[a] for a in C) / Z
        TD = sum(jnp.exp(m[a] - mstar) * V[a] for a in C) / Z
        Zo = sum(jnp.exp(mo[a] - mostar) * so[a] for a in C)
        lse_d = mstar + jnp.log(Z)
        lse_o = mostar + jnp.log(Zo)
        P = TO - lse_o
        T = TD - lse_d
        nC = len(C) * D
        return jnp.sum(-P - (T - P) / nC)

    x1 = _xc((0,))
    x2 = _xc((1,))
    x3 = _xc((2,))
    x13 = _xc((0, 2))
    x23 = _xc((1, 2))
    x12 = _xc((0, 1))
    x123 = _xc((0, 1, 2))

    vals = (mse_p, s1, s2, s3, s12, s13, s23, s123,
            x1, x2, x3, x13, x23, x12, x123)
    lane = jax.lax.broadcasted_iota(jnp.int32, (1, 128), 1)
    out_v = jnp.zeros((1, 128), f32)
    for k, v in enumerate(vals):
        out_v = jnp.where(lane == k, v, out_v)
    o_ref[...] = out_v.reshape(1, 1, 128)


def kernel(data, data1, data2, data3, output1, output2, output3, output):
    B, D = data1.shape
    f32 = jnp.float32

    mb = min(_MIN_BLOCK, B)
    nb1 = B // mb
    mins = pl.pallas_call(
        _min_kernel,
        out_shape=jax.ShapeDtypeStruct((nb1, 1, 128), f32),
        grid=(nb1,),
        in_specs=[pl.BlockSpec((mb, D), lambda i: (i, 0))] * 3,
        out_specs=pl.BlockSpec((1, 1, 128), lambda i: (i, 0, 0)),
        compiler_params=pltpu.CompilerParams(
            dimension_semantics=("parallel",)),
        name="sae_loss_mins",
    )(data1, data2, data3)
    lowers = jnp.floor(jnp.min(mins[:, 0, :3], axis=0))

    R = _ROWS_PER_BLOCK
    nb2 = B // R
    part = pl.pallas_call(
        _main_kernel,
        out_shape=jax.ShapeDtypeStruct((nb2, 1, 128), f32),
        grid=(nb2,),
        in_specs=[
            pl.BlockSpec((R, D), lambda i: (i, 0)),
            pl.BlockSpec((R, D), lambda i: (i, 0)),
            pl.BlockSpec((R, D), lambda i: (i, 0)),
            pl.BlockSpec((R, D), lambda i: (i, 0)),
            pl.BlockSpec((R, D), lambda i: (i, 0)),
            pl.BlockSpec((R, D), lambda i: (i, 0)),
            pl.BlockSpec((R, 3 * D), lambda i: (i, 0)),
            pl.BlockSpec((R, 3 * D), lambda i: (i, 0)),
            pl.BlockSpec(memory_space=pltpu.SMEM),
        ],
        out_specs=pl.BlockSpec((1, 1, 128), lambda i: (i, 0, 0)),
        compiler_params=pltpu.CompilerParams(
            dimension_semantics=("parallel",),
            vmem_limit_bytes=56 * 1024 * 1024),
        name="sae_loss_main",
    )(data1, data2, data3, output1, output2, output3, data, output, lowers)

    sums = jnp.sum(part[:, 0, :], axis=0)
    (mse_s, s1, s2, s3, s12, s13, s23, s123,
     x1, x2, x3, x13, x23, x12, x123) = [sums[k] for k in range(15)]

    n = f32(D)
    logn = jnp.log(n)
    Bf = f32(B)

    mse = 0.5 * mse_s / (Bf * 3 * n)
    H_d1 = logn - s1 / (Bf * n)
    H_d2 = logn - s2 / (Bf * n)
    H_d3 = logn - s3 / (Bf * n)
    H_in13 = logn - s13 / (Bf * n)
    H_in23 = logn - s23 / (Bf * n)
    H_in12 = logn - s12 / (Bf * n)

    H_o1 = x1 / Bf
    H_o2 = x2 / Bf
    H_o3 = x3 / Bf
    H_o13 = x13 / Bf
    H_o23 = x23 / Bf
    H_o12 = x12 / Bf
    H_o123 = x123 / Bf

    H_1 = H_d1 - H_o1
    H_2 = H_d2 - H_o2
    H_3 = H_d3 - H_o3

    H_MI13 = (H_o1 + H_o3 - H_o13) - (H_d1 + H_d3 - H_in13)
    H_MI23 = (H_o2 + H_o3 - H_o23) - (H_d2 + H_d3 - H_in23)
    H_MI12 = (H_o1 + H_o2 - H_o12) - (H_d1 + H_d2 - H_in12)

    data_mu = (s3 + s123 - s13 - s23) / n
    label_cmi = H_o23 - H_o3 + H_o13 - H_o123
    CMI = label_cmi - data_mu

    return 0.9 * mse + 0.1 * (H_1 ** 2 + H_2 ** 2 + H_3 ** 2
                              + H_MI13 ** 2 + H_MI23 ** 2 + H_MI12 ** 2
                              + CMI ** 2)
